# Initial kernel scaffold; baseline (speedup 1.0000x reference)
import functools

import jax
import jax.numpy as jnp
from jax import lax
from jax.experimental import pallas as pl
from jax.experimental.pallas import tpu as pltpu
from jax.experimental.pallas import tpu_sc as plsc

MDIM = 64
CHUNK = 512


def kernel(speaker_id, table):
    orig_shape = speaker_id.shape
    idx_flat = speaker_id.reshape(-1).astype(jnp.int32)
    B = idx_flat.shape[0]

    info = plsc.get_sparse_core_info()
    nc, ns = info.num_cores, info.num_subcores
    nw = nc * ns
    b_per_w = B // nw
    n_chunks = b_per_w // CHUNK
    assert b_per_w % CHUNK == 0 and B % nw == 0

    mesh = plsc.VectorSubcoreMesh(core_axis_name="c", subcore_axis_name="s")

    @functools.partial(
        pl.kernel,
        out_type=jax.ShapeDtypeStruct((B, MDIM), jnp.float32),
        mesh=mesh,
        scratch_types=[
            pltpu.VMEM((CHUNK,), jnp.int32),
            pltpu.VMEM((CHUNK, MDIM), jnp.float32),
            pltpu.SemaphoreType.DMA,
        ],
    )
    def lookup(idx_hbm, table_hbm, out_hbm, idx_v, rows_v, sem):
        wid = lax.axis_index("s") * nc + lax.axis_index("c")
        base = wid * b_per_w

        def body(g, carry):
            off = base + g * CHUNK
            pltpu.sync_copy(idx_hbm.at[pl.ds(off, CHUNK)], idx_v)
            pltpu.async_copy(table_hbm.at[idx_v], rows_v, sem).wait()
            pltpu.sync_copy(rows_v, out_hbm.at[pl.ds(off, CHUNK)])
            return carry

        lax.fori_loop(0, n_chunks, body, 0)

    out = lookup(idx_flat, table)
    return out.reshape(*orig_shape, MDIM)


# speedup vs baseline: 2.4079x; 2.4079x over previous
import functools

import jax
import jax.numpy as jnp
from jax import lax
from jax.experimental import pallas as pl
from jax.experimental.pallas import tpu as pltpu
from jax.experimental.pallas import tpu_sc as plsc

MDIM = 64
CHUNK = 128
MAGIC = 0x01020408


def kernel(speaker_id, table):
    orig_shape = speaker_id.shape
    idx_flat = speaker_id.reshape(-1)
    B = idx_flat.shape[0]
    BQ = B // 4

    idx_words = lax.bitcast_convert_type(
        idx_flat.astype(jnp.int8).reshape(-1, 4), jnp.int32
    )

    qi = jnp.arange(16)
    quad_table = jnp.concatenate(
        [table[(qi >> j) & 1] for j in range(4)], axis=1
    )

    info = plsc.get_sparse_core_info()
    nc, ns = info.num_cores, info.num_subcores
    nw = nc * ns
    q_per_w = BQ // nw
    n_chunks = q_per_w // CHUNK
    assert BQ % nw == 0 and q_per_w % CHUNK == 0

    mesh = plsc.VectorSubcoreMesh(core_axis_name="c", subcore_axis_name="s")

    @functools.partial(
        pl.kernel,
        out_type=jax.ShapeDtypeStruct((BQ, 4 * MDIM), jnp.float32),
        mesh=mesh,
        scratch_types=[
            pltpu.VMEM((CHUNK,), jnp.int32),
            pltpu.VMEM((CHUNK,), jnp.int32),
            pltpu.VMEM((CHUNK, 4 * MDIM), jnp.float32),
            pltpu.SemaphoreType.DMA,
        ],
    )
    def lookup(words_hbm, table_hbm, out_hbm, words_v, qidx_v, rows_v, sem):
        wid = lax.axis_index("s") * nc + lax.axis_index("c")
        base = wid * q_per_w

        def body(g, carry):
            off = base + g * CHUNK
            pltpu.sync_copy(words_hbm.at[pl.ds(off, CHUNK)], words_v)
            for w in range(CHUNK // 16):
                ids4 = words_v[pl.ds(w * 16, 16)]
                qidx_v[pl.ds(w * 16, 16)] = lax.shift_right_logical(
                    ids4 * MAGIC, 24
                )
            pltpu.async_copy(table_hbm.at[qidx_v], rows_v, sem).wait()
            pltpu.sync_copy(rows_v, out_hbm.at[pl.ds(off, CHUNK)])
            return carry

        lax.fori_loop(0, n_chunks, body, 0)

    out = lookup(idx_words, quad_table)
    return out.reshape(*orig_shape, MDIM)


# speedup vs baseline: 2.4248x; 1.0070x over previous
import functools

import jax
import jax.numpy as jnp
from jax import lax
from jax.experimental import pallas as pl
from jax.experimental.pallas import tpu as pltpu
from jax.experimental.pallas import tpu_sc as plsc

MDIM = 64
QDIM = 4 * MDIM
CHUNK = 128
MAGIC = 0x01020408


def kernel(speaker_id, table):
    orig_shape = speaker_id.shape
    idx_flat = speaker_id.reshape(-1)
    B = idx_flat.shape[0]
    BQ = B // 4

    idx_words = lax.bitcast_convert_type(
        idx_flat.astype(jnp.int8).reshape(-1, 4), jnp.int32
    )

    qi = jnp.arange(16)
    quad_table = jnp.concatenate(
        [table[(qi >> j) & 1] for j in range(4)], axis=1
    )

    info = plsc.get_sparse_core_info()
    nc, ns = info.num_cores, info.num_subcores
    nw = nc * ns
    q_per_w = BQ // nw
    n_chunks = q_per_w // CHUNK
    assert BQ % nw == 0 and q_per_w % CHUNK == 0 and n_chunks % 2 == 0

    mesh = plsc.VectorSubcoreMesh(core_axis_name="c", subcore_axis_name="s")

    @functools.partial(
        pl.kernel,
        out_type=jax.ShapeDtypeStruct((BQ, QDIM), jnp.float32),
        mesh=mesh,
        scratch_types=[
            pltpu.VMEM((CHUNK,), jnp.int32),
            [pltpu.VMEM((CHUNK,), jnp.int32) for _ in range(2)],
            [pltpu.VMEM((CHUNK, QDIM), jnp.float32) for _ in range(2)],
            [pltpu.SemaphoreType.DMA for _ in range(2)],
            [pltpu.SemaphoreType.DMA for _ in range(2)],
        ],
    )
    def lookup(words_hbm, table_hbm, out_hbm, words_v, qidx, rows, gsem, ssem):
        wid = lax.axis_index("s") * nc + lax.axis_index("c")
        base = wid * q_per_w

        def load_and_index(g, b):
            pltpu.sync_copy(words_hbm.at[pl.ds(base + g * CHUNK, CHUNK)], words_v)
            for w in range(CHUNK // 16):
                ids4 = words_v[pl.ds(w * 16, 16)]
                qidx[b][pl.ds(w * 16, 16)] = lax.shift_right_logical(
                    ids4 * MAGIC, 24
                )

        def start_gather(b):
            pltpu.async_copy(table_hbm.at[qidx[b]], rows[b], gsem[b])

        def wait_gather(b):
            pltpu.make_async_copy(table_hbm.at[qidx[b]], rows[b], gsem[b]).wait()

        def start_store(g, b):
            pltpu.async_copy(rows[b], out_hbm.at[pl.ds(base + g * CHUNK, CHUNK)], ssem[b])

        def wait_store(b):
            pltpu.make_async_copy(rows[b], out_hbm.at[pl.ds(base, CHUNK)], ssem[b]).wait()

        load_and_index(0, 0)
        start_gather(0)
        load_and_index(1, 1)
        start_gather(1)
        wait_gather(0)
        start_store(0, 0)

        def body(i, carry):
            g0 = 2 * i
            for b in range(2):
                g = g0 + b
                load_and_index(g, b)
                wait_store(b)
                start_gather(b)
                wait_gather(1 - b)
                start_store(g - 1, 1 - b)
            return carry

        lax.fori_loop(1, n_chunks // 2, body, 0)

        wait_gather(1)
        start_store(n_chunks - 1, 1)
        wait_store(0)
        wait_store(1)

    out = lookup(idx_words, quad_table)
    return out.reshape(*orig_shape, MDIM)


# speedup vs baseline: 5.6815x; 2.3430x over previous
import functools

import jax
import jax.numpy as jnp
from jax import lax
from jax.experimental import pallas as pl
from jax.experimental.pallas import tpu as pltpu
from jax.experimental.pallas import tpu_sc as plsc

MDIM = 64
NV = MDIM // 16
WORDS_PER_CHUNK = 64
CELLS_PER_CHUNK = 4 * WORDS_PER_CHUNK


def kernel(speaker_id, table):
    R, C = speaker_id.shape
    ncells = R * C

    idx_words = lax.bitcast_convert_type(
        speaker_id.reshape(-1).astype(jnp.int8).reshape(-1, 4), jnp.int32
    )

    info = plsc.get_sparse_core_info()
    nc, ns = info.num_cores, info.num_subcores
    nsub = nc * ns
    cells_per_w = ncells // nsub
    n_chunks = cells_per_w // CELLS_PER_CHUNK
    assert ncells % nsub == 0 and cells_per_w % CELLS_PER_CHUNK == 0
    assert n_chunks % 2 == 0 and MDIM % 16 == 0

    mesh = plsc.VectorSubcoreMesh(core_axis_name="c", subcore_axis_name="s")

    @functools.partial(
        pl.kernel,
        out_type=jax.ShapeDtypeStruct((R, C, MDIM), jnp.float32),
        mesh=mesh,
        scratch_types=[
            pltpu.VMEM((2, MDIM), jnp.float32),
            pltpu.VMEM((WORDS_PER_CHUNK,), jnp.int32),
            [pltpu.VMEM((CELLS_PER_CHUNK, MDIM), jnp.float32) for _ in range(2)],
            [pltpu.SemaphoreType.DMA for _ in range(2)],
        ],
    )
    def lookup(words_hbm, table_hbm, out_hbm, table_v, words_v, rows, ssem):
        wid = lax.axis_index("s") * nc + lax.axis_index("c")
        wbase = wid * (cells_per_w // 4)
        cbase = wid * cells_per_w

        out_flat = out_hbm.reshape(R * C, MDIM)

        pltpu.sync_copy(table_hbm, table_v)
        t0 = [table_v[0, pl.ds(m * 16, 16)] for m in range(NV)]
        t1 = [table_v[1, pl.ds(m * 16, 16)] for m in range(NV)]

        def expand(g, b):
            pltpu.sync_copy(
                words_hbm.at[pl.ds(wbase + g * WORDS_PER_CHUNK, WORDS_PER_CHUNK)],
                words_v,
            )

            def block_body(blk, carry):
                wv = words_v[pl.ds(blk * 16, 16)]
                cell0 = blk * 64
                for widx in range(16):
                    w = wv[widx]
                    for k in range(4):
                        bit = lax.shift_right_logical(w, 8 * k) & 1
                        cell = cell0 + widx * 4 + k
                        for m in range(NV):
                            rows[b][cell, pl.ds(m * 16, 16)] = jnp.where(
                                bit == 1, t1[m], t0[m]
                            )
                return carry

            lax.fori_loop(0, WORDS_PER_CHUNK // 16, block_body, 0)

        def store_descr(g, b):
            return pltpu.make_async_copy(
                rows[b],
                out_flat.at[pl.ds(cbase + g * CELLS_PER_CHUNK, CELLS_PER_CHUNK)],
                ssem[b],
            )

        expand(0, 0)
        store_descr(0, 0).start()
        expand(1, 1)
        store_descr(1, 1).start()

        def body(it, carry):
            g0 = 2 * it
            for b in range(2):
                g = g0 + b
                store_descr(g, b).wait()
                expand(g, b)
                store_descr(g, b).start()
            return carry

        lax.fori_loop(1, n_chunks // 2, body, 0)

        store_descr(0, 0).wait()
        store_descr(0, 1).wait()

    return lookup(idx_words, table)


# speedup vs baseline: 5.6942x; 1.0022x over previous
import functools

import jax
import jax.numpy as jnp
from jax import lax
from jax.experimental import pallas as pl
from jax.experimental.pallas import tpu as pltpu
from jax.experimental.pallas import tpu_sc as plsc

MDIM = 64
NV = MDIM // 16
WORDS_PER_CHUNK = 64
CELLS_PER_CHUNK = 4 * WORDS_PER_CHUNK


def kernel(speaker_id, table):
    R, C = speaker_id.shape
    ncells = R * C

    idx_words = lax.bitcast_convert_type(
        speaker_id.reshape(-1).astype(jnp.int8).reshape(-1, 4), jnp.int32
    )

    info = plsc.get_sparse_core_info()
    nc, ns = info.num_cores, info.num_subcores
    nsub = nc * ns
    cells_per_w = ncells // nsub
    n_chunks = cells_per_w // CELLS_PER_CHUNK
    assert ncells % nsub == 0 and cells_per_w % CELLS_PER_CHUNK == 0
    assert n_chunks % 2 == 0 and MDIM % 16 == 0

    mesh = plsc.VectorSubcoreMesh(core_axis_name="c", subcore_axis_name="s")

    @functools.partial(
        pl.kernel,
        out_type=jax.ShapeDtypeStruct((R, C, MDIM), jnp.float32),
        mesh=mesh,
        compiler_params=pltpu.CompilerParams(use_tc_tiling_on_sc=True),
        scratch_types=[
            pltpu.VMEM((2, MDIM), jnp.float32),
            pltpu.VMEM((WORDS_PER_CHUNK,), jnp.int32),
            [pltpu.VMEM((CELLS_PER_CHUNK, MDIM), jnp.float32) for _ in range(2)],
            [pltpu.SemaphoreType.DMA for _ in range(2)],
        ],
    )
    def lookup(words_hbm, table_hbm, out_hbm, table_v, words_v, rows, ssem):
        wid = lax.axis_index("s") * nc + lax.axis_index("c")
        wbase = wid * (cells_per_w // 4)
        cbase = wid * cells_per_w

        out_flat = out_hbm.reshape(R * C, MDIM)

        pltpu.sync_copy(table_hbm, table_v)
        t0 = [table_v[0, pl.ds(m * 16, 16)] for m in range(NV)]
        t1 = [table_v[1, pl.ds(m * 16, 16)] for m in range(NV)]

        def expand(g, b):
            pltpu.sync_copy(
                words_hbm.at[pl.ds(wbase + g * WORDS_PER_CHUNK, WORDS_PER_CHUNK)],
                words_v,
            )

            def block_body(blk, carry):
                wv = words_v[pl.ds(blk * 16, 16)]
                cell0 = blk * 64
                for widx in range(16):
                    w = wv[widx]
                    for k in range(4):
                        bit = lax.shift_right_logical(w, 8 * k) & 1
                        cell = cell0 + widx * 4 + k
                        for m in range(NV):
                            rows[b][cell, pl.ds(m * 16, 16)] = jnp.where(
                                bit == 1, t1[m], t0[m]
                            )
                return carry

            lax.fori_loop(0, WORDS_PER_CHUNK // 16, block_body, 0)

        def store_descr(g, b):
            return pltpu.make_async_copy(
                rows[b],
                out_flat.at[pl.ds(cbase + g * CELLS_PER_CHUNK, CELLS_PER_CHUNK)],
                ssem[b],
            )

        expand(0, 0)
        store_descr(0, 0).start()
        expand(1, 1)
        store_descr(1, 1).start()

        def body(it, carry):
            g0 = 2 * it
            for b in range(2):
                g = g0 + b
                store_descr(g, b).wait()
                expand(g, b)
                store_descr(g, b).start()
            return carry

        lax.fori_loop(1, n_chunks // 2, body, 0)

        store_descr(0, 0).wait()
        store_descr(0, 1).wait()

    return lookup(idx_words, table)
